# baseline (speedup 1.0000x reference)
import functools

import jax
import jax.numpy as jnp
from jax.experimental import pallas as pl

_N = 4096
_MAX_PED = 16
_MIN_R = 0.5
_MAX_R = 10.0
_BR = 256


def _body(xrow, yrow, xcol, ycol, hxcol, hycol, idxcol, radcol, out_ref):
    xi = xcol[...]
    yi = ycol[...]
    hx = hxcol[...]
    hy = hycol[...]
    sqh = hx * hx + hy * hy
    xj = xrow[...]
    yj = yrow[...]
    dx = xj - xi
    dy = yj - yi
    squ = dx * dx + dy * dy
    dot = dx * hx + dy * hy
    in_sight = (dot > 0.0) & (4.0 * (dot * dot) > squ * sqh)
    inf = jnp.float32(jnp.inf)
    msk = jnp.where(in_sight, squ, inf)

    bits = jax.lax.bitcast_convert_type(msk, jnp.int32)
    inf_bits = jnp.int32(0x7F800000)
    shift = jnp.int32(-(2**31))
    ms = []
    base = jnp.zeros((_BR, 1), jnp.int32)
    for _ in range(_MAX_PED + 1):
        base_s = base + shift
        d = jnp.min(bits - base_s, axis=1, keepdims=True)
        nxt_bits = d + base_s
        ms.append(jax.lax.bitcast_convert_type(nxt_bits, jnp.float32))
        base = jnp.minimum(nxt_bits + 1, inf_bits)

    s15 = jnp.full((_BR, 1), inf)
    s16 = jnp.full((_BR, 1), inf)
    cum = jnp.zeros((_BR, 1), jnp.int32)
    for k in range(_MAX_PED + 1):
        c = jnp.sum((msk == ms[k]).astype(jnp.int32), axis=1, keepdims=True)
        nxt = cum + c
        s15 = jnp.where((cum <= _MAX_PED - 1) & (nxt > _MAX_PED - 1), ms[k], s15)
        s16 = jnp.where((cum <= _MAX_PED) & (nxt > _MAX_PED), ms[k], s16)
        cum = nxt

    r = 0.5 * (jnp.sqrt(s15) + jnp.sqrt(s16))
    r = jnp.clip(r, _MIN_R, _MAX_R)
    out_ref[...] = jnp.where(idxcol[...] != 0, r, radcol[...])


def kernel(past_ped_positions, ped_positions, indexes, all_radii):
    n = ped_positions.shape[0]
    x = ped_positions[:, 0]
    y = ped_positions[:, 1]
    hx = x - past_ped_positions[:, 0]
    hy = y - past_ped_positions[:, 1]
    xrow = x.reshape(1, n)
    yrow = y.reshape(1, n)
    xcol = x.reshape(n, 1)
    ycol = y.reshape(n, 1)
    hxcol = hx.reshape(n, 1)
    hycol = hy.reshape(n, 1)
    idxcol = indexes.astype(jnp.int32).reshape(n, 1)
    radcol = all_radii.reshape(n, 1)

    grid = (n // _BR,)
    row_spec = pl.BlockSpec((1, n), lambda i: (0, 0))
    col_spec = pl.BlockSpec((_BR, 1), lambda i: (i, 0))
    out = pl.pallas_call(
        _body,
        grid=grid,
        in_specs=[row_spec, row_spec, col_spec, col_spec, col_spec, col_spec,
                  col_spec, col_spec],
        out_specs=col_spec,
        out_shape=jax.ShapeDtypeStruct((n, 1), jnp.float32),
    )(xrow, yrow, xcol, ycol, hxcol, hycol, idxcol, radcol)
    return out.reshape(n)


# speedup vs baseline: 1.0919x; 1.0919x over previous
import functools

import jax
import jax.numpy as jnp
from jax.experimental import pallas as pl

_N = 4096
_MAX_PED = 16
_MIN_R = 0.5
_MAX_R = 10.0
_BR = 256


def _body(xrow, yrow, xcol, ycol, hxcol, hycol, idxcol, radcol, out_ref):
    xi = xcol[...]
    yi = ycol[...]
    hx = hxcol[...]
    hy = hycol[...]
    sqh = hx * hx + hy * hy
    xj = xrow[...]
    yj = yrow[...]
    dx = xj - xi
    dy = yj - yi
    squ = dx * dx + dy * dy
    dot = dx * hx + dy * hy
    in_sight = (dot > 0.0) & (4.0 * (dot * dot) > squ * sqh)
    inf = jnp.float32(jnp.inf)
    msk = jnp.where(in_sight, squ, inf)

    ms = []
    m = jnp.full((_BR, 1), -jnp.inf, jnp.float32)
    for _ in range(_MAX_PED + 1):
        m = jnp.min(jnp.where(msk > m, msk, inf), axis=1, keepdims=True)
        ms.append(m)

    s15 = jnp.full((_BR, 1), inf)
    s16 = jnp.full((_BR, 1), inf)
    cum = jnp.zeros((_BR, 1), jnp.int32)
    for k in range(_MAX_PED + 1):
        c = jnp.sum((msk == ms[k]).astype(jnp.int32), axis=1, keepdims=True)
        nxt = cum + c
        s15 = jnp.where((cum <= _MAX_PED - 1) & (nxt > _MAX_PED - 1), ms[k], s15)
        s16 = jnp.where((cum <= _MAX_PED) & (nxt > _MAX_PED), ms[k], s16)
        cum = nxt

    r = 0.5 * (jnp.sqrt(s15) + jnp.sqrt(s16))
    r = jnp.clip(r, _MIN_R, _MAX_R)
    out_ref[...] = jnp.where(idxcol[...] != 0, r, radcol[...])


def kernel(past_ped_positions, ped_positions, indexes, all_radii):
    n = ped_positions.shape[0]
    x = ped_positions[:, 0]
    y = ped_positions[:, 1]
    hx = x - past_ped_positions[:, 0]
    hy = y - past_ped_positions[:, 1]
    xrow = x.reshape(1, n)
    yrow = y.reshape(1, n)
    xcol = x.reshape(n, 1)
    ycol = y.reshape(n, 1)
    hxcol = hx.reshape(n, 1)
    hycol = hy.reshape(n, 1)
    idxcol = indexes.astype(jnp.int32).reshape(n, 1)
    radcol = all_radii.reshape(n, 1)

    grid = (n // _BR,)
    row_spec = pl.BlockSpec((1, n), lambda i: (0, 0))
    col_spec = pl.BlockSpec((_BR, 1), lambda i: (i, 0))
    out = pl.pallas_call(
        _body,
        grid=grid,
        in_specs=[row_spec, row_spec, col_spec, col_spec, col_spec, col_spec,
                  col_spec, col_spec],
        out_specs=col_spec,
        out_shape=jax.ShapeDtypeStruct((n, 1), jnp.float32),
    )(xrow, yrow, xcol, ycol, hxcol, hycol, idxcol, radcol)
    return out.reshape(n)


# speedup vs baseline: 1.7790x; 1.6292x over previous
import functools

import jax
import jax.numpy as jnp
from jax.experimental import pallas as pl

_N = 4096
_MAX_PED = 16
_MIN_R = 0.5
_MAX_R = 10.0
_BR = 256


def _body(xrow, yrow, xcol, ycol, hxcol, hycol, idxcol, radcol, out_ref):
    xi = xcol[...]
    yi = ycol[...]
    hx = hxcol[...]
    hy = hycol[...]
    sqh = hx * hx + hy * hy
    xj = xrow[...]
    yj = yrow[...]
    dx = xj - xi
    dy = yj - yi
    squ = dx * dx + dy * dy
    dot = dx * hx + dy * hy
    in_sight = (dot > 0.0) & (4.0 * (dot * dot) > squ * sqh)
    inf = jnp.float32(jnp.inf)
    msk = jnp.where(in_sight, squ, inf)

    ms = []
    m = jnp.full((_BR, 1), -jnp.inf, jnp.float32)
    for _ in range(_MAX_PED + 1):
        m = jnp.min(jnp.where(msk > m, msk, inf), axis=1, keepdims=True)
        ms.append(m)

    cnt17 = jnp.sum((msk <= ms[_MAX_PED]).astype(jnp.int32), axis=1,
                    keepdims=True)
    ties = jnp.any((ms[_MAX_PED] < inf) & (cnt17 > _MAX_PED + 1))

    def fast(_):
        return ms[_MAX_PED - 1], ms[_MAX_PED]

    def slow(_):
        s15 = jnp.full((_BR, 1), inf)
        s16 = jnp.full((_BR, 1), inf)
        cum = jnp.zeros((_BR, 1), jnp.int32)
        for k in range(_MAX_PED + 1):
            c = jnp.sum((msk == ms[k]).astype(jnp.int32), axis=1,
                        keepdims=True)
            nxt = cum + c
            s15 = jnp.where((cum <= _MAX_PED - 1) & (nxt > _MAX_PED - 1),
                            ms[k], s15)
            s16 = jnp.where((cum <= _MAX_PED) & (nxt > _MAX_PED), ms[k], s16)
            cum = nxt
        return s15, s16

    s15, s16 = jax.lax.cond(ties, slow, fast, None)

    r = 0.5 * (jnp.sqrt(s15) + jnp.sqrt(s16))
    r = jnp.clip(r, _MIN_R, _MAX_R)
    out_ref[...] = jnp.where(idxcol[...] != 0, r, radcol[...])


def kernel(past_ped_positions, ped_positions, indexes, all_radii):
    n = ped_positions.shape[0]
    x = ped_positions[:, 0]
    y = ped_positions[:, 1]
    hx = x - past_ped_positions[:, 0]
    hy = y - past_ped_positions[:, 1]
    xrow = x.reshape(1, n)
    yrow = y.reshape(1, n)
    xcol = x.reshape(n, 1)
    ycol = y.reshape(n, 1)
    hxcol = hx.reshape(n, 1)
    hycol = hy.reshape(n, 1)
    idxcol = indexes.astype(jnp.int32).reshape(n, 1)
    radcol = all_radii.reshape(n, 1)

    grid = (n // _BR,)
    row_spec = pl.BlockSpec((1, n), lambda i: (0, 0))
    col_spec = pl.BlockSpec((_BR, 1), lambda i: (i, 0))
    out = pl.pallas_call(
        _body,
        grid=grid,
        in_specs=[row_spec, row_spec, col_spec, col_spec, col_spec, col_spec,
                  col_spec, col_spec],
        out_specs=col_spec,
        out_shape=jax.ShapeDtypeStruct((n, 1), jnp.float32),
    )(xrow, yrow, xcol, ycol, hxcol, hycol, idxcol, radcol)
    return out.reshape(n)
